# baseline (speedup 1.0000x reference)
import functools

import jax
import jax.numpy as jnp
from jax import lax
from jax.experimental import pallas as pl
from jax.experimental.pallas import tpu as pltpu
from jax.experimental.pallas import tpu_sc as plsc

N_V = 1000
N_D = 64
B = 4096
T = 200
NW = 32
PER_B = B // NW
NBUF = 4
ROUNDS = PER_B // NBUF


@jax.jit
def _sc_embedding_lookup(idx, table):
  mesh = plsc.VectorSubcoreMesh(core_axis_name="c", subcore_axis_name="s")

  @functools.partial(
      pl.kernel,
      mesh=mesh,
      out_type=jax.ShapeDtypeStruct((B, T, N_D), jnp.float32),
      compiler_params=pltpu.CompilerParams(use_tc_tiling_on_sc=False),
      scratch_types=(
          [pltpu.VMEM((PER_B, T), jnp.int32)]
          + [pltpu.VMEM((T, N_D), jnp.float32) for _ in range(NBUF)]
          + [pltpu.SemaphoreType.DMA for _ in range(2 * NBUF)]
      ),
  )
  def k(idx_hbm, table_hbm, out_hbm, idx_v, *bufs_and_sems):
    rows = bufs_and_sems[:NBUF]
    gsems = bufs_and_sems[NBUF:2 * NBUF]
    ssems = bufs_and_sems[2 * NBUF:]

    wid = lax.axis_index("s") * 2 + lax.axis_index("c")
    b0 = wid * PER_B

    pltpu.sync_copy(idx_hbm.at[pl.ds(b0, PER_B)], idx_v)

    for b in range(NBUF):
      pltpu.async_copy(table_hbm.at[idx_v.at[b]], rows[b], gsems[b])

    def round_body(r, carry):
      j0 = r * NBUF
      for b in range(NBUF):
        j = j0 + b
        pltpu.make_async_copy(
            table_hbm.at[idx_v.at[j]], rows[b], gsems[b]).wait()
        pltpu.async_copy(rows[b], out_hbm.at[b0 + j], ssems[b])
      for b in range(NBUF):
        j = j0 + b
        pltpu.make_async_copy(
            rows[b], out_hbm.at[b0 + j], ssems[b]).wait()
        nj = j + NBUF

        @pl.when(nj < PER_B)
        def _():
          pltpu.async_copy(table_hbm.at[idx_v.at[nj]], rows[b], gsems[b])

      return carry

    lax.fori_loop(0, ROUNDS, round_body, 0)

  return k(idx, table)


def kernel(input_, table):
  return _sc_embedding_lookup(input_, table)
